# baseline (speedup 1.0000x reference)
import functools

import jax
import jax.numpy as jnp
from jax import lax
from jax.experimental import pallas as pl
from jax.experimental.pallas import tpu as pltpu
from jax.experimental.pallas import tpu_sc as plsc

BN = 8192
NQ = 2048
K = 16
H = 8
DH = 64
D = 512
AB = 512
QB = 256
CB = 128
NW = 32
ROWS = BN * K
RPW = ROWS // NW
CHUNK = 128
FW = 128


def _prep_body(x_ref, p_ref, g_ref, Wqkv_ref, Wgeo_ref, bgeo_ref, Wpos_ref,
               q_ref, kv_ref, f_ref):
    x = x_ref[...]
    qkv = jnp.dot(x, Wqkv_ref[...], preferred_element_type=jnp.float32)
    q_ref[...] = qkv[:, :D]
    kb = qkv[:, D:2 * D].astype(jnp.bfloat16).astype(jnp.float32)
    vb = qkv[:, 2 * D:].astype(jnp.bfloat16).astype(jnp.float32)
    ki = lax.bitcast_convert_type(kb, jnp.int32)
    vi = lax.bitcast_convert_type(vb, jnp.int32)
    kv_ref[...] = jnp.bitwise_or(lax.shift_right_logical(ki, 16), vi)
    a = jnp.dot(p_ref[...], Wpos_ref[...], preferred_element_type=jnp.float32)
    geo = g_ref[...]
    gw = jnp.dot(geo, Wgeo_ref[...], preferred_element_type=jnp.float32) + bgeo_ref[...]
    n3 = geo[:, :3]
    nrm = jnp.sqrt(jnp.sum(n3 * n3, axis=-1, keepdims=True))
    gn = n3 / jnp.maximum(nrm, 1e-8)
    pad = jnp.zeros((AB, FW - 75), jnp.float32)
    f_ref[...] = jnp.concatenate([a, gn, gw, pad], axis=-1)


def _topk_body(pq_ref, pT_ref, idx_ref):
    b = pl.program_id(0)
    pq = pq_ref[0]
    sq = jnp.zeros((QB, NQ), jnp.float32)
    for c in range(3):
        dc = pq[:, c:c + 1] - pT_ref[0, c:c + 1, :]
        sq = sq + dc * dc
    iota = lax.broadcasted_iota(jnp.int32, (1, NQ), 1)
    cols = []
    for _ in range(K):
        m = jnp.min(sq, axis=-1, keepdims=True)
        am = jnp.min(jnp.where(sq == m, iota, NQ), axis=-1)
        cols.append(am)
        sq = jnp.where(iota == am[:, None], jnp.inf, sq)
    idx = jnp.stack(cols, axis=-1) + b * NQ
    idx_ref[0] = idx.astype(jnp.int32)


def _attn_body(q_ref, fq_ref, kvg_ref, fg_ref, Wp_ref, bpos_ref, bproj_ref,
               seg_ref, exp_ref, o_ref):
    scale = float(D) ** -0.5
    R = CB * K
    q = q_ref[...]
    fq = fq_ref[...]
    kv = kvg_ref[...]
    k_rows = lax.bitcast_convert_type(lax.shift_left(kv, 16), jnp.float32)
    v_rows = lax.bitcast_convert_type(
        jnp.bitwise_and(kv, jnp.int32(-65536)), jnp.float32)
    fg = fg_ref[...]

    q_rep = jnp.broadcast_to(q[:, None, :], (CB, K, D)).reshape(R, D)
    fq_rep = jnp.broadcast_to(fq[:, None, :], (CB, K, FW)).reshape(R, FW)

    pos = jax.nn.relu(fq_rep[:, 0:64] - fg[:, 0:64] + bpos_ref[...])
    pos_rep = jnp.broadcast_to(pos[:, None, :], (R, H, DH)).reshape(R, D)

    cos = jnp.sum(fq_rep[:, 64:67] * fg[:, 64:67], axis=-1, keepdims=True)

    m = scale * k_rows + 0.5 * pos_rep
    dots = jnp.dot(q_rep * m, seg_ref[...], preferred_element_type=jnp.float32)
    dots = dots + 0.3 * cos * fq_rep[:, 67:75]

    d3 = dots.reshape(CB, K, H)
    mx = jnp.max(d3, axis=1, keepdims=True)
    e = jnp.exp(d3 - mx)
    attn = (e / jnp.sum(e, axis=1, keepdims=True)).reshape(R, H)

    attn_rep = jnp.dot(attn, exp_ref[...], preferred_element_type=jnp.float32)
    of = attn_rep * (v_rows + pos_rep)
    out = jnp.sum(of.reshape(CB, K, D), axis=1)
    o_ref[...] = (jnp.dot(out, Wp_ref[...], preferred_element_type=jnp.float32)
                  + bproj_ref[...])


def _make_sc_gather():
    mesh = plsc.VectorSubcoreMesh(core_axis_name="c", subcore_axis_name="s")

    @functools.partial(
        pl.kernel, mesh=mesh,
        out_type=(jax.ShapeDtypeStruct((ROWS, D), jnp.int32),
                  jax.ShapeDtypeStruct((ROWS, FW), jnp.float32)),
        scratch_types=[
            pltpu.VMEM((CHUNK,), jnp.int32),
            pltpu.VMEM((CHUNK, D), jnp.int32),
            pltpu.VMEM((CHUNK, FW), jnp.float32),
            pltpu.SemaphoreType.DMA,
            pltpu.SemaphoreType.DMA,
        ],
    )
    def sc_gather(kv_hbm, f_hbm, idx_hbm, kv_out, f_out, idx_v, kv_v, f_v, s1, s2):
        wid = lax.axis_index("s") * 2 + lax.axis_index("c")
        base = wid * RPW

        def body(i, carry):
            off = base + i * CHUNK
            pltpu.sync_copy(idx_hbm.at[pl.ds(off, CHUNK)], idx_v)
            c1 = pltpu.async_copy(kv_hbm.at[idx_v], kv_v, s1)
            c2 = pltpu.async_copy(f_hbm.at[idx_v], f_v, s2)
            c1.wait()
            c2.wait()
            pltpu.sync_copy(kv_v, kv_out.at[pl.ds(off, CHUNK)])
            pltpu.sync_copy(f_v, f_out.at[pl.ds(off, CHUNK)])
            return carry

        lax.fori_loop(0, RPW // CHUNK, body, 0)

    return sc_gather


_sc_gather_built = None


def _sc_gather(kv_tab, f_tab, idx_flat):
    global _sc_gather_built
    if _sc_gather_built is None:
        _sc_gather_built = _make_sc_gather()
    return _sc_gather_built(kv_tab, f_tab, idx_flat)


def _prep_call(x2, p2, g2, W_qkv, W_geo, b_geo, W_pos):
    grid = (BN // AB,)
    return pl.pallas_call(
        _prep_body,
        grid=grid,
        in_specs=[
            pl.BlockSpec((AB, D), lambda i: (i, 0)),
            pl.BlockSpec((AB, 3), lambda i: (i, 0)),
            pl.BlockSpec((AB, 4), lambda i: (i, 0)),
            pl.BlockSpec((D, 3 * D), lambda i: (0, 0)),
            pl.BlockSpec((4, H), lambda i: (0, 0)),
            pl.BlockSpec((1, H), lambda i: (0, 0)),
            pl.BlockSpec((3, DH), lambda i: (0, 0)),
        ],
        out_specs=[
            pl.BlockSpec((AB, D), lambda i: (i, 0)),
            pl.BlockSpec((AB, D), lambda i: (i, 0)),
            pl.BlockSpec((AB, FW), lambda i: (i, 0)),
        ],
        out_shape=[
            jax.ShapeDtypeStruct((BN, D), jnp.float32),
            jax.ShapeDtypeStruct((BN, D), jnp.int32),
            jax.ShapeDtypeStruct((BN, FW), jnp.float32),
        ],
    )(x2, p2, g2, W_qkv, W_geo, b_geo.reshape(1, H), W_pos)


def _topk_call(points, pT):
    grid = (4, NQ // QB)
    return pl.pallas_call(
        _topk_body,
        grid=grid,
        in_specs=[
            pl.BlockSpec((1, QB, 3), lambda b, i: (b, i, 0)),
            pl.BlockSpec((1, 3, NQ), lambda b, i: (b, 0, 0)),
        ],
        out_specs=pl.BlockSpec((1, QB, K), lambda b, i: (b, i, 0)),
        out_shape=jax.ShapeDtypeStruct((4, NQ, K), jnp.int32),
    )(points, pT)


def _attn_call(q_tab, f_tab, kv_g, f_g, W_proj, b_pos, b_proj):
    grid = (BN // CB,)
    lane_head = jnp.arange(D, dtype=jnp.int32) // DH
    seg = (lane_head[:, None] == jnp.arange(H, dtype=jnp.int32)[None, :]
           ).astype(jnp.float32)
    expm = seg.T
    return pl.pallas_call(
        _attn_body,
        grid=grid,
        in_specs=[
            pl.BlockSpec((CB, D), lambda i: (i, 0)),
            pl.BlockSpec((CB, FW), lambda i: (i, 0)),
            pl.BlockSpec((CB * K, D), lambda i: (i, 0)),
            pl.BlockSpec((CB * K, FW), lambda i: (i, 0)),
            pl.BlockSpec((D, D), lambda i: (0, 0)),
            pl.BlockSpec((1, DH), lambda i: (0, 0)),
            pl.BlockSpec((1, D), lambda i: (0, 0)),
            pl.BlockSpec((D, H), lambda i: (0, 0)),
            pl.BlockSpec((H, D), lambda i: (0, 0)),
        ],
        out_specs=pl.BlockSpec((CB, D), lambda i: (i, 0)),
        out_shape=jax.ShapeDtypeStruct((BN, D), jnp.float32),
    )(q_tab, f_tab, kv_g, f_g, W_proj, b_pos, b_proj, seg, expm)


def kernel(x, points, geo_features, W_qkv, W_geo, b_geo, W_pos, b_pos, W_proj, b_proj):
    b, n, d = x.shape
    x2 = x.reshape(BN, D)
    p2 = points.reshape(BN, 3)
    g2 = geo_features.reshape(BN, 4)
    q_tab, kv_tab, f_tab = _prep_call(x2, p2, g2, W_qkv, W_geo, b_geo, W_pos)
    pT = points.transpose(0, 2, 1)
    idx = _topk_call(points, pT)
    idx_flat = idx.reshape(ROWS)
    kv_g, f_g = _sc_gather(kv_tab, f_tab, idx_flat)
    out = _attn_call(q_tab, f_tab, kv_g, f_g, W_proj,
                     b_pos.reshape(1, DH), b_proj.reshape(1, D))
    return out.reshape(b, n, d)


# speedup vs baseline: 1.0130x; 1.0130x over previous
import functools

import jax
import jax.numpy as jnp
from jax import lax
from jax.experimental import pallas as pl
from jax.experimental.pallas import tpu as pltpu
from jax.experimental.pallas import tpu_sc as plsc

BN = 8192
NQ = 2048
K = 16
H = 8
DH = 64
D = 512
AB = 512
QB = 256
CB = 128
NW = 32
ROWS = BN * K
RPW = ROWS // NW
CHUNK = 64
FW = 128


def _prep_body(x_ref, p_ref, g_ref, Wqkv_ref, Wgeo_ref, bgeo_ref, Wpos_ref,
               q_ref, kv_ref, f_ref):
    x = x_ref[...]
    qkv = jnp.dot(x, Wqkv_ref[...], preferred_element_type=jnp.float32)
    q_ref[...] = qkv[:, :D]
    kb = qkv[:, D:2 * D].astype(jnp.bfloat16).astype(jnp.float32)
    vb = qkv[:, 2 * D:].astype(jnp.bfloat16).astype(jnp.float32)
    ki = lax.bitcast_convert_type(kb, jnp.int32)
    vi = lax.bitcast_convert_type(vb, jnp.int32)
    kv_ref[...] = jnp.bitwise_or(lax.shift_right_logical(ki, 16), vi)
    a = jnp.dot(p_ref[...], Wpos_ref[...], preferred_element_type=jnp.float32)
    geo = g_ref[...]
    gw = jnp.dot(geo, Wgeo_ref[...], preferred_element_type=jnp.float32) + bgeo_ref[...]
    n3 = geo[:, :3]
    nrm = jnp.sqrt(jnp.sum(n3 * n3, axis=-1, keepdims=True))
    gn = n3 / jnp.maximum(nrm, 1e-8)
    pad = jnp.zeros((AB, FW - 75), jnp.float32)
    f_ref[...] = jnp.concatenate([a, gn, gw, pad], axis=-1)


def _topk_body(pq_ref, pT_ref, idx_ref):
    b = pl.program_id(0)
    pq = pq_ref[0]
    sq = jnp.zeros((QB, NQ), jnp.float32)
    for c in range(3):
        dc = pq[:, c:c + 1] - pT_ref[0, c:c + 1, :]
        sq = sq + dc * dc
    iota = lax.broadcasted_iota(jnp.int32, (1, NQ), 1)
    cols = []
    for _ in range(K):
        m = jnp.min(sq, axis=-1, keepdims=True)
        am = jnp.min(jnp.where(sq == m, iota, NQ), axis=-1)
        cols.append(am)
        sq = jnp.where(iota == am[:, None], jnp.inf, sq)
    idx = jnp.stack(cols, axis=-1) + b * NQ
    idx_ref[0] = idx.astype(jnp.int32)


def _attn_body(q_ref, fq_ref, kvg_ref, fg_ref, Wp_ref, bpos_ref, bproj_ref,
               seg_ref, exp_ref, o_ref):
    scale = float(D) ** -0.5
    R = CB * K
    q = q_ref[...]
    fq = fq_ref[...]
    kv = kvg_ref[...]
    k_rows = lax.bitcast_convert_type(lax.shift_left(kv, 16), jnp.float32)
    v_rows = lax.bitcast_convert_type(
        jnp.bitwise_and(kv, jnp.int32(-65536)), jnp.float32)
    fg = fg_ref[...]

    q_rep = jnp.broadcast_to(q[:, None, :], (CB, K, D)).reshape(R, D)
    fq_rep = jnp.broadcast_to(fq[:, None, :], (CB, K, FW)).reshape(R, FW)

    pos = jax.nn.relu(fq_rep[:, 0:64] - fg[:, 0:64] + bpos_ref[...])
    pos_rep = jnp.broadcast_to(pos[:, None, :], (R, H, DH)).reshape(R, D)

    cos = jnp.sum(fq_rep[:, 64:67] * fg[:, 64:67], axis=-1, keepdims=True)

    m = scale * k_rows + 0.5 * pos_rep
    dots = jnp.dot((q_rep * m).astype(jnp.bfloat16), seg_ref[...],
                   preferred_element_type=jnp.float32)
    dots = dots + 0.3 * cos * fq_rep[:, 67:75]

    d3 = dots.reshape(CB, K, H)
    mx = jnp.max(d3, axis=1, keepdims=True)
    e = jnp.exp(d3 - mx)
    attn = (e / jnp.sum(e, axis=1, keepdims=True)).reshape(R, H)

    attn_rep = jnp.dot(attn, exp_ref[...], preferred_element_type=jnp.float32)
    of = attn_rep * (v_rows + pos_rep)
    out = jnp.sum(of.reshape(CB, K, D), axis=1)
    o_ref[...] = (jnp.dot(out.astype(jnp.bfloat16),
                          Wp_ref[...].astype(jnp.bfloat16),
                          preferred_element_type=jnp.float32)
                  + bproj_ref[...])


def _make_sc_gather():
    mesh = plsc.VectorSubcoreMesh(core_axis_name="c", subcore_axis_name="s")

    @functools.partial(
        pl.kernel, mesh=mesh,
        out_type=(jax.ShapeDtypeStruct((ROWS, D), jnp.int32),
                  jax.ShapeDtypeStruct((ROWS, FW), jnp.float32)),
        scratch_types=[
            pltpu.VMEM((RPW,), jnp.int32),
            pltpu.VMEM((CHUNK, D), jnp.int32),
            pltpu.VMEM((CHUNK, D), jnp.int32),
            pltpu.VMEM((CHUNK, FW), jnp.float32),
            pltpu.VMEM((CHUNK, FW), jnp.float32),
            pltpu.SemaphoreType.DMA,
            pltpu.SemaphoreType.DMA,
        ],
    )
    def sc_gather(kv_hbm, f_hbm, idx_hbm, kv_out, f_out,
                  idx_all, kv0, kv1, f0, f1, s0, s1):
        wid = lax.axis_index("s") * 2 + lax.axis_index("c")
        base = wid * RPW
        pltpu.sync_copy(idx_hbm.at[pl.ds(base, RPW)], idx_all)

        def body(p, carry):
            l0 = 2 * p * CHUNK
            l1 = l0 + CHUNK
            ck0 = pltpu.async_copy(kv_hbm.at[idx_all.at[pl.ds(l0, CHUNK)]], kv0, s0)
            cf0 = pltpu.async_copy(f_hbm.at[idx_all.at[pl.ds(l0, CHUNK)]], f0, s0)
            ck1 = pltpu.async_copy(kv_hbm.at[idx_all.at[pl.ds(l1, CHUNK)]], kv1, s1)
            cf1 = pltpu.async_copy(f_hbm.at[idx_all.at[pl.ds(l1, CHUNK)]], f1, s1)
            ck0.wait()
            cf0.wait()
            pltpu.sync_copy(kv0, kv_out.at[pl.ds(base + l0, CHUNK)])
            pltpu.sync_copy(f0, f_out.at[pl.ds(base + l0, CHUNK)])
            ck1.wait()
            cf1.wait()
            pltpu.sync_copy(kv1, kv_out.at[pl.ds(base + l1, CHUNK)])
            pltpu.sync_copy(f1, f_out.at[pl.ds(base + l1, CHUNK)])
            return carry

        lax.fori_loop(0, RPW // (2 * CHUNK), body, 0)

    return sc_gather


_sc_gather_built = None


def _sc_gather(kv_tab, f_tab, idx_flat):
    global _sc_gather_built
    if _sc_gather_built is None:
        _sc_gather_built = _make_sc_gather()
    return _sc_gather_built(kv_tab, f_tab, idx_flat)


def _prep_call(x2, p2, g2, W_qkv, W_geo, b_geo, W_pos):
    grid = (BN // AB,)
    return pl.pallas_call(
        _prep_body,
        grid=grid,
        in_specs=[
            pl.BlockSpec((AB, D), lambda i: (i, 0)),
            pl.BlockSpec((AB, 3), lambda i: (i, 0)),
            pl.BlockSpec((AB, 4), lambda i: (i, 0)),
            pl.BlockSpec((D, 3 * D), lambda i: (0, 0)),
            pl.BlockSpec((4, H), lambda i: (0, 0)),
            pl.BlockSpec((1, H), lambda i: (0, 0)),
            pl.BlockSpec((3, DH), lambda i: (0, 0)),
        ],
        out_specs=[
            pl.BlockSpec((AB, D), lambda i: (i, 0)),
            pl.BlockSpec((AB, D), lambda i: (i, 0)),
            pl.BlockSpec((AB, FW), lambda i: (i, 0)),
        ],
        out_shape=[
            jax.ShapeDtypeStruct((BN, D), jnp.float32),
            jax.ShapeDtypeStruct((BN, D), jnp.int32),
            jax.ShapeDtypeStruct((BN, FW), jnp.float32),
        ],
    )(x2, p2, g2, W_qkv, W_geo, b_geo.reshape(1, H), W_pos)


def _topk_call(points, pT):
    grid = (4, NQ // QB)
    return pl.pallas_call(
        _topk_body,
        grid=grid,
        in_specs=[
            pl.BlockSpec((1, QB, 3), lambda b, i: (b, i, 0)),
            pl.BlockSpec((1, 3, NQ), lambda b, i: (b, 0, 0)),
        ],
        out_specs=pl.BlockSpec((1, QB, K), lambda b, i: (b, i, 0)),
        out_shape=jax.ShapeDtypeStruct((4, NQ, K), jnp.int32),
    )(points, pT)


def _attn_call(q_tab, f_tab, kv_g, f_g, W_proj, b_pos, b_proj):
    grid = (BN // CB,)
    lane_head = jnp.arange(D, dtype=jnp.int32) // DH
    seg = (lane_head[:, None] == jnp.arange(H, dtype=jnp.int32)[None, :]
           ).astype(jnp.bfloat16)
    expm = seg.T.astype(jnp.float32)
    return pl.pallas_call(
        _attn_body,
        grid=grid,
        in_specs=[
            pl.BlockSpec((CB, D), lambda i: (i, 0)),
            pl.BlockSpec((CB, FW), lambda i: (i, 0)),
            pl.BlockSpec((CB * K, D), lambda i: (i, 0)),
            pl.BlockSpec((CB * K, FW), lambda i: (i, 0)),
            pl.BlockSpec((D, D), lambda i: (0, 0)),
            pl.BlockSpec((1, DH), lambda i: (0, 0)),
            pl.BlockSpec((1, D), lambda i: (0, 0)),
            pl.BlockSpec((D, H), lambda i: (0, 0)),
            pl.BlockSpec((H, D), lambda i: (0, 0)),
        ],
        out_specs=pl.BlockSpec((CB, D), lambda i: (i, 0)),
        out_shape=jax.ShapeDtypeStruct((BN, D), jnp.float32),
    )(q_tab, f_tab, kv_g, f_g, W_proj, b_pos, b_proj, seg, expm)


def kernel(x, points, geo_features, W_qkv, W_geo, b_geo, W_pos, b_pos, W_proj, b_proj):
    b, n, d = x.shape
    x2 = x.reshape(BN, D)
    p2 = points.reshape(BN, 3)
    g2 = geo_features.reshape(BN, 4)
    q_tab, kv_tab, f_tab = _prep_call(x2, p2, g2, W_qkv, W_geo, b_geo, W_pos)
    pT = points.transpose(0, 2, 1)
    idx = _topk_call(points, pT)
    idx_flat = idx.reshape(ROWS)
    kv_g, f_g = _sc_gather(kv_tab, f_tab, idx_flat)
    out = _attn_call(q_tab, f_tab, kv_g, f_g, W_proj,
                     b_pos.reshape(1, DH), b_proj.reshape(1, D))
    return out.reshape(b, n, d)


# speedup vs baseline: 1.4691x; 1.4502x over previous
import functools

import jax
import jax.numpy as jnp
from jax import lax
from jax.experimental import pallas as pl
from jax.experimental.pallas import tpu as pltpu
from jax.experimental.pallas import tpu_sc as plsc

BN = 8192
NQ = 2048
K = 16
H = 8
DH = 64
D = 512
AB = 512
QB = 256
CB = 128
NW = 32
ROWS = BN * K
RPW = ROWS // NW
CHUNK = 64
FW = 128


def _prep_body(x_ref, p_ref, g_ref, Wqkv_ref, Wgeo_ref, bgeo_ref, Wpos_ref,
               q_ref, kv_ref, f_ref):
    x = x_ref[...]
    qkv = jnp.dot(x, Wqkv_ref[...], preferred_element_type=jnp.float32)
    q_ref[...] = qkv[:, :D]
    kb = qkv[:, D:2 * D].astype(jnp.bfloat16).astype(jnp.float32)
    vb = qkv[:, 2 * D:].astype(jnp.bfloat16).astype(jnp.float32)
    ki = lax.bitcast_convert_type(kb, jnp.int32)
    vi = lax.bitcast_convert_type(vb, jnp.int32)
    kv_ref[...] = jnp.bitwise_or(lax.shift_right_logical(ki, 16), vi)
    a = jnp.dot(p_ref[...], Wpos_ref[...], preferred_element_type=jnp.float32)
    geo = g_ref[...]
    gw = jnp.dot(geo, Wgeo_ref[...], preferred_element_type=jnp.float32) + bgeo_ref[...]
    n3 = geo[:, :3]
    nrm = jnp.sqrt(jnp.sum(n3 * n3, axis=-1, keepdims=True))
    gn = n3 / jnp.maximum(nrm, 1e-8)
    pad = jnp.zeros((AB, FW - 75), jnp.float32)
    f_ref[...] = jnp.concatenate([a, gn, gw, pad], axis=-1)


def _topk_body(pq_ref, pT_ref, idx_ref):
    b = pl.program_id(0)
    pq = pq_ref[0]
    sq = jnp.zeros((QB, NQ), jnp.float32)
    for c in range(3):
        dc = pq[:, c:c + 1] - pT_ref[0, c:c + 1, :]
        sq = sq + dc * dc
    iota = lax.broadcasted_iota(jnp.int32, (1, NQ), 1)
    cols = []
    for _ in range(K):
        m = jnp.min(sq, axis=-1, keepdims=True)
        am = jnp.min(jnp.where(sq == m, iota, NQ), axis=-1)
        cols.append(am)
        sq = jnp.where(iota == am[:, None], jnp.inf, sq)
    idx = jnp.stack(cols, axis=-1) + b * NQ
    idx_ref[0] = idx.astype(jnp.int32)


def _attn_body(q_ref, fq_ref, kvg_ref, fg_ref, Wp_ref, bpos_ref, bproj_ref,
               seg_ref, exp_ref, tile_ref, sumk_ref, o_ref):
    scale = float(D) ** -0.5
    R = CB * K
    q = q_ref[...]
    fq = fq_ref[...]
    kv = kvg_ref[...]
    k_rows = lax.bitcast_convert_type(lax.shift_left(kv, 16), jnp.float32)
    v_rows = lax.bitcast_convert_type(
        jnp.bitwise_and(kv, jnp.int32(-65536)), jnp.float32)
    fg = fg_ref[...]

    q_rep = jnp.broadcast_to(q[:, None, :], (CB, K, D)).reshape(R, D)
    fq_rep = jnp.broadcast_to(fq[:, None, :], (CB, K, FW)).reshape(R, FW)

    pos = jax.nn.relu(fq_rep[:, 0:64] - fg[:, 0:64] + bpos_ref[...])
    pos_rep = jnp.dot(pos.astype(jnp.bfloat16), tile_ref[...],
                      preferred_element_type=jnp.float32)

    cos = jnp.sum(fq_rep[:, 64:67] * fg[:, 64:67], axis=-1, keepdims=True)

    m = scale * k_rows + 0.5 * pos_rep
    dots = jnp.dot((q_rep * m).astype(jnp.bfloat16), seg_ref[...],
                   preferred_element_type=jnp.float32)
    dots = dots + 0.3 * cos * fq_rep[:, 67:75]

    d3 = dots.reshape(CB, K, H)
    mx = jnp.max(d3, axis=1, keepdims=True)
    e = jnp.exp(d3 - mx)
    attn = (e / jnp.sum(e, axis=1, keepdims=True)).reshape(R, H)

    attn_rep = jnp.dot(attn, exp_ref[...], preferred_element_type=jnp.float32)
    of = attn_rep * (v_rows + pos_rep)
    out = jnp.dot(sumk_ref[...], of.astype(jnp.bfloat16),
                  preferred_element_type=jnp.float32)
    o_ref[...] = (jnp.dot(out.astype(jnp.bfloat16),
                          Wp_ref[...].astype(jnp.bfloat16),
                          preferred_element_type=jnp.float32)
                  + bproj_ref[...])


def _make_sc_gather(rows):
    rpw = rows // NW
    mesh = plsc.VectorSubcoreMesh(core_axis_name="c", subcore_axis_name="s")

    @functools.partial(
        pl.kernel, mesh=mesh,
        out_type=(jax.ShapeDtypeStruct((rows, D), jnp.int32),
                  jax.ShapeDtypeStruct((rows, FW), jnp.float32)),
        scratch_types=[
            pltpu.VMEM((rpw,), jnp.int32),
            pltpu.VMEM((CHUNK, D), jnp.int32),
            pltpu.VMEM((CHUNK, D), jnp.int32),
            pltpu.VMEM((CHUNK, FW), jnp.float32),
            pltpu.VMEM((CHUNK, FW), jnp.float32),
            pltpu.SemaphoreType.DMA,
            pltpu.SemaphoreType.DMA,
        ],
    )
    def sc_gather(kv_hbm, f_hbm, idx_hbm, kv_out, f_out,
                  idx_all, kv0, kv1, f0, f1, s0, s1):
        wid = lax.axis_index("s") * 2 + lax.axis_index("c")
        base = wid * rpw
        pltpu.sync_copy(idx_hbm.at[pl.ds(base, rpw)], idx_all)

        def body(p, carry):
            l0 = 2 * p * CHUNK
            l1 = l0 + CHUNK
            ck0 = pltpu.async_copy(kv_hbm.at[idx_all.at[pl.ds(l0, CHUNK)]], kv0, s0)
            cf0 = pltpu.async_copy(f_hbm.at[idx_all.at[pl.ds(l0, CHUNK)]], f0, s0)
            ck1 = pltpu.async_copy(kv_hbm.at[idx_all.at[pl.ds(l1, CHUNK)]], kv1, s1)
            cf1 = pltpu.async_copy(f_hbm.at[idx_all.at[pl.ds(l1, CHUNK)]], f1, s1)
            ck0.wait()
            cf0.wait()
            pltpu.sync_copy(kv0, kv_out.at[pl.ds(base + l0, CHUNK)])
            pltpu.sync_copy(f0, f_out.at[pl.ds(base + l0, CHUNK)])
            ck1.wait()
            cf1.wait()
            pltpu.sync_copy(kv1, kv_out.at[pl.ds(base + l1, CHUNK)])
            pltpu.sync_copy(f1, f_out.at[pl.ds(base + l1, CHUNK)])
            return carry

        lax.fori_loop(0, rpw // (2 * CHUNK), body, 0)

    return sc_gather


_sc_gather_built = {}


def _sc_gather(kv_tab, f_tab, idx_flat):
    rows = idx_flat.shape[0]
    if rows not in _sc_gather_built:
        _sc_gather_built[rows] = _make_sc_gather(rows)
    return _sc_gather_built[rows](kv_tab, f_tab, idx_flat)


def _prep_call(x2, p2, g2, W_qkv, W_geo, b_geo, W_pos):
    grid = (BN // AB,)
    return pl.pallas_call(
        _prep_body,
        grid=grid,
        in_specs=[
            pl.BlockSpec((AB, D), lambda i: (i, 0)),
            pl.BlockSpec((AB, 3), lambda i: (i, 0)),
            pl.BlockSpec((AB, 4), lambda i: (i, 0)),
            pl.BlockSpec((D, 3 * D), lambda i: (0, 0)),
            pl.BlockSpec((4, H), lambda i: (0, 0)),
            pl.BlockSpec((1, H), lambda i: (0, 0)),
            pl.BlockSpec((3, DH), lambda i: (0, 0)),
        ],
        out_specs=[
            pl.BlockSpec((AB, D), lambda i: (i, 0)),
            pl.BlockSpec((AB, D), lambda i: (i, 0)),
            pl.BlockSpec((AB, FW), lambda i: (i, 0)),
        ],
        out_shape=[
            jax.ShapeDtypeStruct((BN, D), jnp.float32),
            jax.ShapeDtypeStruct((BN, D), jnp.int32),
            jax.ShapeDtypeStruct((BN, FW), jnp.float32),
        ],
    )(x2, p2, g2, W_qkv, W_geo, b_geo.reshape(1, H), W_pos)


def _topk_call(points, pT):
    nb = points.shape[0]
    grid = (nb, NQ // QB)
    return pl.pallas_call(
        _topk_body,
        grid=grid,
        in_specs=[
            pl.BlockSpec((1, QB, 3), lambda b, i: (b, i, 0)),
            pl.BlockSpec((1, 3, NQ), lambda b, i: (b, 0, 0)),
        ],
        out_specs=pl.BlockSpec((1, QB, K), lambda b, i: (b, i, 0)),
        out_shape=jax.ShapeDtypeStruct((nb, NQ, K), jnp.int32),
    )(points, pT)


def _attn_consts():
    lane_head = jnp.arange(D, dtype=jnp.int32) // DH
    seg = (lane_head[:, None] == jnp.arange(H, dtype=jnp.int32)[None, :]
           ).astype(jnp.bfloat16)
    expm = seg.T.astype(jnp.float32)
    lane = jnp.arange(D, dtype=jnp.int32) % DH
    tile = (jnp.arange(DH, dtype=jnp.int32)[:, None] == lane[None, :]
            ).astype(jnp.bfloat16)
    row_q = jnp.arange(CB * K, dtype=jnp.int32) // K
    sumk = (jnp.arange(CB, dtype=jnp.int32)[:, None] == row_q[None, :]
            ).astype(jnp.bfloat16)
    return seg, expm, tile, sumk


def _attn_call(q_tab, f_tab, kv_g, f_g, W_proj, b_pos, b_proj, consts, boff):
    rows = kv_g.shape[0]
    nq_rows = rows // K
    grid = (nq_rows // CB,)
    seg, expm, tile, sumk = consts
    return pl.pallas_call(
        _attn_body,
        grid=grid,
        in_specs=[
            pl.BlockSpec((CB, D), lambda i, _b=boff: (_b + i, 0)),
            pl.BlockSpec((CB, FW), lambda i, _b=boff: (_b + i, 0)),
            pl.BlockSpec((CB * K, D), lambda i: (i, 0)),
            pl.BlockSpec((CB * K, FW), lambda i: (i, 0)),
            pl.BlockSpec((D, D), lambda i: (0, 0)),
            pl.BlockSpec((1, DH), lambda i: (0, 0)),
            pl.BlockSpec((1, D), lambda i: (0, 0)),
            pl.BlockSpec((D, H), lambda i: (0, 0)),
            pl.BlockSpec((H, D), lambda i: (0, 0)),
            pl.BlockSpec((DH, D), lambda i: (0, 0)),
            pl.BlockSpec((CB, CB * K), lambda i: (0, 0)),
        ],
        out_specs=pl.BlockSpec((CB, D), lambda i: (i, 0)),
        out_shape=jax.ShapeDtypeStruct((nq_rows, D), jnp.float32),
    )(q_tab, f_tab, kv_g, f_g, W_proj, b_pos, b_proj, seg, expm, tile, sumk)


def kernel(x, points, geo_features, W_qkv, W_geo, b_geo, W_pos, b_pos, W_proj, b_proj):
    b, n, d = x.shape
    x2 = x.reshape(BN, D)
    p2 = points.reshape(BN, 3)
    g2 = geo_features.reshape(BN, 4)
    q_tab, kv_tab, f_tab = _prep_call(x2, p2, g2, W_qkv, W_geo, b_geo, W_pos)
    pT = points.transpose(0, 2, 1)
    consts = _attn_consts()
    bp = b_pos.reshape(1, DH)
    bpr = b_proj.reshape(1, D)
    outs = []
    for bb in range(b):
        idx_b = _topk_call(points[bb:bb + 1], pT[bb:bb + 1])
        idx_flat = idx_b.reshape(n * K) + jnp.int32(bb * NQ)
        kv_g, f_g = _sc_gather(kv_tab, f_tab, idx_flat)
        outs.append(_attn_call(q_tab, f_tab, kv_g, f_g, W_proj, bp, bpr,
                               consts, boff=bb * (NQ // CB)))
    return jnp.concatenate(outs, axis=0).reshape(b, n, d)


# speedup vs baseline: 1.5455x; 1.0521x over previous
import functools

import jax
import jax.numpy as jnp
from jax import lax
from jax.experimental import pallas as pl
from jax.experimental.pallas import tpu as pltpu
from jax.experimental.pallas import tpu_sc as plsc

BN = 8192
NQ = 2048
K = 16
H = 8
DH = 64
D = 512
AB = 512
QB = 256
CB = 128
NW = 32
ROWS = BN * K
RPW = ROWS // NW
CHUNK = 64
FW = 128


def _prep_body(x_ref, p_ref, g_ref, Wqkv_ref, Wgeo_ref, bgeo_ref, Wpos_ref,
               q_ref, kv_ref, f_ref):
    x = x_ref[...]
    qkv = jnp.dot(x, Wqkv_ref[...], preferred_element_type=jnp.float32)
    q_ref[...] = qkv[:, :D]
    kb = qkv[:, D:2 * D].astype(jnp.bfloat16).astype(jnp.float32)
    vb = qkv[:, 2 * D:].astype(jnp.bfloat16).astype(jnp.float32)
    ki = lax.bitcast_convert_type(kb, jnp.int32)
    vi = lax.bitcast_convert_type(vb, jnp.int32)
    kv_ref[...] = jnp.bitwise_or(lax.shift_right_logical(ki, 16), vi)
    a = jnp.dot(p_ref[...], Wpos_ref[...], preferred_element_type=jnp.float32)
    geo = g_ref[...]
    gw = jnp.dot(geo, Wgeo_ref[...], preferred_element_type=jnp.float32) + bgeo_ref[...]
    n3 = geo[:, :3]
    nrm = jnp.sqrt(jnp.sum(n3 * n3, axis=-1, keepdims=True))
    gn = n3 / jnp.maximum(nrm, 1e-8)
    pad = jnp.zeros((AB, FW - 75), jnp.float32)
    f_ref[...] = jnp.concatenate([a, gn, gw, pad], axis=-1)


def _topk_body(pq_ref, pT_ref, idx_ref):
    b = pl.program_id(0)
    i = pl.program_id(1)
    pq = pq_ref[0]
    sq = jnp.zeros((QB, NQ), jnp.float32)
    for c in range(3):
        dc = pq[:, c:c + 1] - pT_ref[0, c:c + 1, :]
        sq = sq + dc * dc
    iota = lax.broadcasted_iota(jnp.int32, (1, NQ), 1)
    rows = i * QB + lax.broadcasted_iota(jnp.int32, (QB,), 0)
    sq = jnp.where(iota == rows[:, None], jnp.inf, sq)
    cols = [rows]
    for _ in range(K - 1):
        m = jnp.min(sq, axis=-1, keepdims=True)
        eq = sq == m
        am = jnp.min(jnp.where(eq, iota, NQ), axis=-1)
        cols.append(am)
        sq = jnp.where(eq, jnp.inf, sq)
    idx = jnp.stack(cols, axis=-1) + b * NQ
    idx_ref[0] = idx.astype(jnp.int32)


def _attn_body(q_ref, fq_ref, kvg_ref, fg_ref, Wp_ref, bpos_ref, bproj_ref,
               seg_ref, exp_ref, tile_ref, sumk_ref, o_ref):
    scale = float(D) ** -0.5
    R = CB * K
    q = q_ref[...]
    fq = fq_ref[...]
    kv = kvg_ref[...]
    k_rows = lax.bitcast_convert_type(lax.shift_left(kv, 16), jnp.float32)
    v_rows = lax.bitcast_convert_type(
        jnp.bitwise_and(kv, jnp.int32(-65536)), jnp.float32)
    fg = fg_ref[...]

    q_rep = jnp.broadcast_to(q[:, None, :], (CB, K, D)).reshape(R, D)
    fq_rep = jnp.broadcast_to(fq[:, None, :], (CB, K, FW)).reshape(R, FW)

    pos = jax.nn.relu(fq_rep[:, 0:64] - fg[:, 0:64] + bpos_ref[...])
    pos_rep = jnp.dot(pos.astype(jnp.bfloat16), tile_ref[...],
                      preferred_element_type=jnp.float32)

    cos = jnp.sum(fq_rep[:, 64:67] * fg[:, 64:67], axis=-1, keepdims=True)

    m = scale * k_rows + 0.5 * pos_rep
    dots = jnp.dot((q_rep * m).astype(jnp.bfloat16), seg_ref[...],
                   preferred_element_type=jnp.float32)
    dots = dots + 0.3 * cos * fq_rep[:, 67:75]

    d3 = dots.reshape(CB, K, H)
    mx = jnp.max(d3, axis=1, keepdims=True)
    e = jnp.exp(d3 - mx)
    attn = (e / jnp.sum(e, axis=1, keepdims=True)).reshape(R, H)

    attn_rep = jnp.dot(attn, exp_ref[...], preferred_element_type=jnp.float32)
    of = attn_rep * (v_rows + pos_rep)
    out = jnp.dot(sumk_ref[...], of.astype(jnp.bfloat16),
                  preferred_element_type=jnp.float32)
    o_ref[...] = (jnp.dot(out.astype(jnp.bfloat16),
                          Wp_ref[...].astype(jnp.bfloat16),
                          preferred_element_type=jnp.float32)
                  + bproj_ref[...])


def _make_sc_gather(rows):
    rpw = rows // NW
    mesh = plsc.VectorSubcoreMesh(core_axis_name="c", subcore_axis_name="s")

    @functools.partial(
        pl.kernel, mesh=mesh,
        out_type=(jax.ShapeDtypeStruct((rows, D), jnp.int32),
                  jax.ShapeDtypeStruct((rows, FW), jnp.float32)),
        scratch_types=[
            pltpu.VMEM((rpw,), jnp.int32),
            pltpu.VMEM((CHUNK, D), jnp.int32),
            pltpu.VMEM((CHUNK, D), jnp.int32),
            pltpu.VMEM((CHUNK, FW), jnp.float32),
            pltpu.VMEM((CHUNK, FW), jnp.float32),
            pltpu.SemaphoreType.DMA,
            pltpu.SemaphoreType.DMA,
        ],
    )
    def sc_gather(kv_hbm, f_hbm, idx_hbm, kv_out, f_out,
                  idx_all, kv0, kv1, f0, f1, s0, s1):
        wid = lax.axis_index("s") * 2 + lax.axis_index("c")
        base = wid * rpw
        pltpu.sync_copy(idx_hbm.at[pl.ds(base, rpw)], idx_all)

        def body(p, carry):
            l0 = 2 * p * CHUNK
            l1 = l0 + CHUNK
            ck0 = pltpu.async_copy(kv_hbm.at[idx_all.at[pl.ds(l0, CHUNK)]], kv0, s0)
            cf0 = pltpu.async_copy(f_hbm.at[idx_all.at[pl.ds(l0, CHUNK)]], f0, s0)
            ck1 = pltpu.async_copy(kv_hbm.at[idx_all.at[pl.ds(l1, CHUNK)]], kv1, s1)
            cf1 = pltpu.async_copy(f_hbm.at[idx_all.at[pl.ds(l1, CHUNK)]], f1, s1)
            ck0.wait()
            cf0.wait()
            pltpu.sync_copy(kv0, kv_out.at[pl.ds(base + l0, CHUNK)])
            pltpu.sync_copy(f0, f_out.at[pl.ds(base + l0, CHUNK)])
            ck1.wait()
            cf1.wait()
            pltpu.sync_copy(kv1, kv_out.at[pl.ds(base + l1, CHUNK)])
            pltpu.sync_copy(f1, f_out.at[pl.ds(base + l1, CHUNK)])
            return carry

        lax.fori_loop(0, rpw // (2 * CHUNK), body, 0)

    return sc_gather


_sc_gather_built = {}


def _sc_gather(kv_tab, f_tab, idx_flat):
    rows = idx_flat.shape[0]
    if rows not in _sc_gather_built:
        _sc_gather_built[rows] = _make_sc_gather(rows)
    return _sc_gather_built[rows](kv_tab, f_tab, idx_flat)


def _prep_call(x2, p2, g2, W_qkv, W_geo, b_geo, W_pos):
    grid = (BN // AB,)
    return pl.pallas_call(
        _prep_body,
        grid=grid,
        in_specs=[
            pl.BlockSpec((AB, D), lambda i: (i, 0)),
            pl.BlockSpec((AB, 3), lambda i: (i, 0)),
            pl.BlockSpec((AB, 4), lambda i: (i, 0)),
            pl.BlockSpec((D, 3 * D), lambda i: (0, 0)),
            pl.BlockSpec((4, H), lambda i: (0, 0)),
            pl.BlockSpec((1, H), lambda i: (0, 0)),
            pl.BlockSpec((3, DH), lambda i: (0, 0)),
        ],
        out_specs=[
            pl.BlockSpec((AB, D), lambda i: (i, 0)),
            pl.BlockSpec((AB, D), lambda i: (i, 0)),
            pl.BlockSpec((AB, FW), lambda i: (i, 0)),
        ],
        out_shape=[
            jax.ShapeDtypeStruct((BN, D), jnp.float32),
            jax.ShapeDtypeStruct((BN, D), jnp.int32),
            jax.ShapeDtypeStruct((BN, FW), jnp.float32),
        ],
    )(x2, p2, g2, W_qkv, W_geo, b_geo.reshape(1, H), W_pos)


def _topk_call(points, pT):
    nb = points.shape[0]
    grid = (nb, NQ // QB)
    return pl.pallas_call(
        _topk_body,
        grid=grid,
        in_specs=[
            pl.BlockSpec((1, QB, 3), lambda b, i: (b, i, 0)),
            pl.BlockSpec((1, 3, NQ), lambda b, i: (b, 0, 0)),
        ],
        out_specs=pl.BlockSpec((1, QB, K), lambda b, i: (b, i, 0)),
        out_shape=jax.ShapeDtypeStruct((nb, NQ, K), jnp.int32),
    )(points, pT)


def _attn_consts():
    lane_head = jnp.arange(D, dtype=jnp.int32) // DH
    seg = (lane_head[:, None] == jnp.arange(H, dtype=jnp.int32)[None, :]
           ).astype(jnp.bfloat16)
    expm = seg.T.astype(jnp.float32)
    lane = jnp.arange(D, dtype=jnp.int32) % DH
    tile = (jnp.arange(DH, dtype=jnp.int32)[:, None] == lane[None, :]
            ).astype(jnp.bfloat16)
    row_q = jnp.arange(CB * K, dtype=jnp.int32) // K
    sumk = (jnp.arange(CB, dtype=jnp.int32)[:, None] == row_q[None, :]
            ).astype(jnp.bfloat16)
    return seg, expm, tile, sumk


def _attn_call(q_tab, f_tab, kv_g, f_g, W_proj, b_pos, b_proj, consts, boff):
    rows = kv_g.shape[0]
    nq_rows = rows // K
    grid = (nq_rows // CB,)
    seg, expm, tile, sumk = consts
    return pl.pallas_call(
        _attn_body,
        grid=grid,
        in_specs=[
            pl.BlockSpec((CB, D), lambda i, _b=boff: (_b + i, 0)),
            pl.BlockSpec((CB, FW), lambda i, _b=boff: (_b + i, 0)),
            pl.BlockSpec((CB * K, D), lambda i: (i, 0)),
            pl.BlockSpec((CB * K, FW), lambda i: (i, 0)),
            pl.BlockSpec((D, D), lambda i: (0, 0)),
            pl.BlockSpec((1, DH), lambda i: (0, 0)),
            pl.BlockSpec((1, D), lambda i: (0, 0)),
            pl.BlockSpec((D, H), lambda i: (0, 0)),
            pl.BlockSpec((H, D), lambda i: (0, 0)),
            pl.BlockSpec((DH, D), lambda i: (0, 0)),
            pl.BlockSpec((CB, CB * K), lambda i: (0, 0)),
        ],
        out_specs=pl.BlockSpec((CB, D), lambda i: (i, 0)),
        out_shape=jax.ShapeDtypeStruct((nq_rows, D), jnp.float32),
    )(q_tab, f_tab, kv_g, f_g, W_proj, b_pos, b_proj, seg, expm, tile, sumk)


def kernel(x, points, geo_features, W_qkv, W_geo, b_geo, W_pos, b_pos, W_proj, b_proj):
    b, n, d = x.shape
    x2 = x.reshape(BN, D)
    p2 = points.reshape(BN, 3)
    g2 = geo_features.reshape(BN, 4)
    q_tab, kv_tab, f_tab = _prep_call(x2, p2, g2, W_qkv, W_geo, b_geo, W_pos)
    pT = points.transpose(0, 2, 1)
    consts = _attn_consts()
    bp = b_pos.reshape(1, DH)
    bpr = b_proj.reshape(1, D)
    outs = []
    for bb in range(b):
        idx_b = _topk_call(points[bb:bb + 1], pT[bb:bb + 1])
        idx_flat = idx_b.reshape(n * K) + jnp.int32(bb * NQ)
        kv_g, f_g = _sc_gather(kv_tab, f_tab, idx_flat)
        outs.append(_attn_call(q_tab, f_tab, kv_g, f_g, W_proj, bp, bpr,
                               consts, boff=bb * (NQ // CB)))
    return jnp.concatenate(outs, axis=0).reshape(b, n, d)
